# baseline (speedup 1.0000x reference)
import jax
import jax.numpy as jnp
from jax import lax
from jax.experimental import pallas as pl
from jax.experimental.pallas import tpu as pltpu
from jax.experimental.pallas import tpu_sc as plsc

N = 10000
D = 128
E = 320000
MAXNORM = 1.0 - 1e-3
MIN_NORM = 1e-15
EPS = 1e-7

NC = 2
NS = 16
NH = N // NC
CH = 128
EP = 327680
K = EP // (NS * CH)
NB = 4
BB = 16
NBLK = K // BB
GPB = BB // NB
WB = 312
SENT = 2 ** 30

RB = 1000


def _rownorm(v):
    return jnp.maximum(jnp.sqrt(jnp.sum(v * v, axis=-1, keepdims=True)), MIN_NORM)


def _artanh(y):
    y = jnp.clip(y, -1.0 + EPS, 1.0 - EPS)
    return 0.5 * jnp.log((1.0 + y) / (1.0 - y))


def _proj(h):
    n = _rownorm(h)
    return jnp.where(n > MAXNORM, h / n * MAXNORM, h)


def _logmap0(h):
    pn = _rownorm(h)
    return _artanh(pn) * h / pn


def _expmap0(u):
    r = _rownorm(u)
    return jnp.tanh(r) * u / r


def _linear_body(x_ref, w_ref, o_ref):
    xb = x_ref[...]
    w = w_ref[...]
    mx = lax.dot_general(xb, w, (((1,), (1,)), ((), ())),
                         preferred_element_type=jnp.float32)
    xn = _rownorm(xb)
    mxn = _rownorm(mx)
    res = jnp.tanh(mxn / xn * _artanh(xn)) * mx / mxn
    cond = jnp.all(mx == 0.0, axis=-1, keepdims=True)
    res = jnp.where(cond, 0.0, res)
    o_ref[...] = _logmap0(_proj(res))


def _mid_body(p_ref, o_ref):
    o_ref[...] = _logmap0(_proj(_expmap0(p_ref[...])))


def _final_body(p_ref, o_ref):
    h = _proj(_expmap0(p_ref[...]))
    xt = jnp.maximum(_logmap0(h), 0.0)
    o_ref[...] = _proj(_expmap0(xt))


def _linear(x, w):
    return pl.pallas_call(
        _linear_body,
        grid=(N // RB,),
        in_specs=[
            pl.BlockSpec((RB, D), lambda i: (i, 0)),
            pl.BlockSpec((D, D), lambda i: (0, 0)),
        ],
        out_specs=pl.BlockSpec((RB, D), lambda i: (i, 0)),
        out_shape=jax.ShapeDtypeStruct((N, D), jnp.float32),
    )(x, w)


def _mid(p):
    return pl.pallas_call(
        _mid_body,
        grid=(N // RB,),
        in_specs=[pl.BlockSpec((RB, D), lambda i: (i, 0))],
        out_specs=pl.BlockSpec((RB, D), lambda i: (i, 0)),
        out_shape=jax.ShapeDtypeStruct((N, D), jnp.float32),
    )(p)


def _final(p):
    return pl.pallas_call(
        _final_body,
        grid=(N // RB,),
        in_specs=[pl.BlockSpec((RB, D), lambda i: (i, 0))],
        out_specs=pl.BlockSpec((RB, D), lambda i: (i, 0)),
        out_shape=jax.ShapeDtypeStruct((N, D), jnp.float32),
    )(p)


def _agg_body(xt_hbm, srcm_hbm, dstm_hbm, out_hbm,
              acc, src_b, dst_b,
              rows0, rows1, rows2, rows3,
              isem_s, isem_d,
              gsem0, gsem1, gsem2, gsem3,
              ssem0, ssem1, ssem2, ssem3):
    c = lax.axis_index("c")
    s = lax.axis_index("s")
    rows = (rows0, rows1, rows2, rows3)
    gsem = (gsem0, gsem1, gsem2, gsem3)
    ssem = (ssem0, ssem1, ssem2, ssem3)

    def src_of(j):
        return src_b.at[(j // BB) % 2, j % BB]

    def dst_of(j):
        return dst_b.at[(j // BB) % 2, j % BB]

    def gather_ref(j):
        return xt_hbm.at[plsc.Indices(src_of(j), ignored_value=SENT)]

    def scat_ref(j):
        return acc.at[plsc.Indices(dst_of(j), ignored_value=SENT)]

    def zrow(i, _):
        for k in range(D // 16):
            rows0[i, pl.ds(k * 16, 16)] = jnp.zeros((16,), jnp.float32)
        return 0
    lax.fori_loop(0, CH, zrow, 0)
    for m in range(WB // 104):
        pltpu.sync_copy(rows0.at[pl.ds(0, 104)],
                        acc.at[pl.ds(s * WB + m * 104, 104)])

    @pl.when(s == NS - 1)
    def _():
        pltpu.sync_copy(rows0.at[pl.ds(0, NH - NS * WB)],
                        acc.at[pl.ds(NS * WB, NH - NS * WB)])
    plsc.subcore_barrier()

    pltpu.sync_copy(srcm_hbm.at[c, pl.ds(s * K, BB)], src_b.at[0])
    pltpu.sync_copy(dstm_hbm.at[c, pl.ds(s * K, BB)], dst_b.at[0])

    def pref_src(nb):
        return (srcm_hbm.at[c, pl.ds(s * K + nb * BB, BB)], src_b.at[nb % 2])

    def pref_dst(nb):
        return (dstm_hbm.at[c, pl.ds(s * K + nb * BB, BB)], dst_b.at[nb % 2])

    def gather(j, b):
        pltpu.async_copy(gather_ref(j), rows[b], gsem[b])

    for j in range(NB - 1):
        gather(j, j)

    def group(g, _):
        blk = g // GPB
        pos = g % GPB

        @pl.when((pos == GPB - 1) & (blk + 1 < NBLK))
        def _():
            a, bdst = pref_src(blk + 1)
            pltpu.make_async_copy(a, bdst, isem_s).wait()
            a, bdst = pref_dst(blk + 1)
            pltpu.make_async_copy(a, bdst, isem_d).wait()

        @pl.when((pos == 1) & (blk + 1 < NBLK))
        def _():
            a, bdst = pref_src(blk + 1)
            pltpu.async_copy(a, bdst, isem_s)
            a, bdst = pref_dst(blk + 1)
            pltpu.async_copy(a, bdst, isem_d)

        for b in range(NB):
            j = g * NB + b
            bn = (b + NB - 1) % NB

            @pl.when(j >= 1)
            def _():
                pltpu.make_async_copy(rows[bn], scat_ref(j), ssem[bn]).wait()

            @pl.when(j + NB - 1 < K)
            def _():
                gather(j + NB - 1, bn)
            pltpu.make_async_copy(gather_ref(j), rows[b], gsem[b]).wait()
            pltpu.async_copy(rows[b], scat_ref(j), ssem[b], add=True)
        return 0
    lax.fori_loop(0, K // NB, group, 0)

    pltpu.make_async_copy(rows[(K - 1) % NB], scat_ref(K - 1),
                          ssem[(K - 1) % NB]).wait()

    plsc.subcore_barrier()
    pltpu.sync_copy(acc.at[pl.ds(s * WB, WB)],
                    out_hbm.at[pl.ds(c * NH + s * WB, WB)])

    @pl.when(s == NS - 1)
    def _():
        pltpu.sync_copy(acc.at[pl.ds(NS * WB, NH - NS * WB)],
                        out_hbm.at[pl.ds(c * NH + NS * WB, NH - NS * WB)])


def _aggregate(xt, srcm, dstm):
    mesh = plsc.VectorSubcoreMesh(core_axis_name="c", subcore_axis_name="s")
    f = pl.kernel(
        _agg_body,
        mesh=mesh,
        out_type=jax.ShapeDtypeStruct((N, D), jnp.float32),
        scratch_types=[
            pltpu.VMEM_SHARED((NH, D), jnp.float32),
            pltpu.VMEM((2, BB, CH), jnp.int32),
            pltpu.VMEM((2, BB, CH), jnp.int32),
            pltpu.VMEM((CH, D), jnp.float32),
            pltpu.VMEM((CH, D), jnp.float32),
            pltpu.VMEM((CH, D), jnp.float32),
            pltpu.VMEM((CH, D), jnp.float32),
            pltpu.SemaphoreType.DMA,
            pltpu.SemaphoreType.DMA,
            pltpu.SemaphoreType.DMA,
            pltpu.SemaphoreType.DMA,
            pltpu.SemaphoreType.DMA,
            pltpu.SemaphoreType.DMA,
            pltpu.SemaphoreType.DMA,
            pltpu.SemaphoreType.DMA,
            pltpu.SemaphoreType.DMA,
            pltpu.SemaphoreType.DMA,
        ],
    )
    return f(xt, srcm, dstm)


def kernel(x, edge_index, W):
    src = edge_index[0]
    dst = edge_index[1]
    lo = dst < NH
    srcm = jnp.stack([jnp.where(lo, src, SENT), jnp.where(lo, SENT, src)])
    dstm = jnp.stack([jnp.where(lo, dst, SENT),
                      jnp.where(lo, SENT, dst - NH)])
    pad = jnp.full((2, EP - E), SENT, jnp.int32)
    srcm = jnp.concatenate([srcm, pad], axis=1).reshape(2, EP // CH, CH)
    dstm = jnp.concatenate([dstm, pad], axis=1).reshape(2, EP // CH, CH)

    xt1 = _linear(x, W)
    p = _aggregate(xt1, srcm, dstm)
    xt2 = _mid(p)
    q = _aggregate(xt2, srcm, dstm)
    return _final(q)


# speedup vs baseline: 1.0234x; 1.0234x over previous
import jax
import jax.numpy as jnp
from jax import lax
from jax.experimental import pallas as pl
from jax.experimental.pallas import tpu as pltpu
from jax.experimental.pallas import tpu_sc as plsc

N = 10000
D = 128
E = 320000
MAXNORM = 1.0 - 1e-3
MIN_NORM = 1e-15
EPS = 1e-7

NC = 2
NS = 16
DC = D // NC
CH = 125
K = E // (NS * CH)
NB = 4
WB = 624

RB = 1000


def _rownorm(v):
    return jnp.maximum(jnp.sqrt(jnp.sum(v * v, axis=-1, keepdims=True)), MIN_NORM)


def _artanh(y):
    y = jnp.clip(y, -1.0 + EPS, 1.0 - EPS)
    return 0.5 * jnp.log((1.0 + y) / (1.0 - y))


def _proj(h):
    n = _rownorm(h)
    return jnp.where(n > MAXNORM, h / n * MAXNORM, h)


def _logmap0(h):
    pn = _rownorm(h)
    return _artanh(pn) * h / pn


def _expmap0(u):
    r = _rownorm(u)
    return jnp.tanh(r) * u / r


def _linear_body(x_ref, w_ref, o_ref):
    xb = x_ref[...]
    w = w_ref[...]
    mx = lax.dot_general(xb, w, (((1,), (1,)), ((), ())),
                         preferred_element_type=jnp.float32)
    xn = _rownorm(xb)
    mxn = _rownorm(mx)
    res = jnp.tanh(mxn / xn * _artanh(xn)) * mx / mxn
    cond = jnp.all(mx == 0.0, axis=-1, keepdims=True)
    res = jnp.where(cond, 0.0, res)
    xt = _logmap0(_proj(res))
    o_ref[0] = xt[:, :DC]
    o_ref[1] = xt[:, DC:]


def _mid_body(p_ref, o_ref):
    agg = jnp.concatenate([p_ref[0], p_ref[1]], axis=-1)
    xt = _logmap0(_proj(_expmap0(agg)))
    o_ref[0] = xt[:, :DC]
    o_ref[1] = xt[:, DC:]


def _final_body(p_ref, o_ref):
    agg = jnp.concatenate([p_ref[0], p_ref[1]], axis=-1)
    h = _proj(_expmap0(agg))
    xt = jnp.maximum(_logmap0(h), 0.0)
    o_ref[...] = _proj(_expmap0(xt))


def _linear(x, w):
    return pl.pallas_call(
        _linear_body,
        grid=(N // RB,),
        in_specs=[
            pl.BlockSpec((RB, D), lambda i: (i, 0)),
            pl.BlockSpec((D, D), lambda i: (0, 0)),
        ],
        out_specs=pl.BlockSpec((NC, RB, DC), lambda i: (0, i, 0)),
        out_shape=jax.ShapeDtypeStruct((NC, N, DC), jnp.float32),
    )(x, w)


def _mid(p):
    return pl.pallas_call(
        _mid_body,
        grid=(N // RB,),
        in_specs=[pl.BlockSpec((NC, RB, DC), lambda i: (0, i, 0))],
        out_specs=pl.BlockSpec((NC, RB, DC), lambda i: (0, i, 0)),
        out_shape=jax.ShapeDtypeStruct((NC, N, DC), jnp.float32),
    )(p)


def _final(p):
    return pl.pallas_call(
        _final_body,
        grid=(N // RB,),
        in_specs=[pl.BlockSpec((NC, RB, DC), lambda i: (0, i, 0))],
        out_specs=pl.BlockSpec((RB, D), lambda i: (i, 0)),
        out_shape=jax.ShapeDtypeStruct((N, D), jnp.float32),
    )(p)


def _agg_body(xt_hbm, src_hbm, dst_hbm, out_hbm,
              acc, src_idx, dst_idx,
              rows0, rows1, rows2, rows3,
              gsem0, gsem1, gsem2, gsem3,
              ssem0, ssem1, ssem2, ssem3):
    c = lax.axis_index("c")
    s = lax.axis_index("s")
    rows = (rows0, rows1, rows2, rows3)
    gsem = (gsem0, gsem1, gsem2, gsem3)
    ssem = (ssem0, ssem1, ssem2, ssem3)
    xt_sc = xt_hbm.at[c]

    def zrow(i, _):
        for k in range(DC // 16):
            rows0[i, pl.ds(k * 16, 16)] = jnp.zeros((16,), jnp.float32)
        return 0
    lax.fori_loop(0, CH, zrow, 0)
    for m in range(WB // 104):
        pltpu.sync_copy(rows0.at[pl.ds(0, 104)],
                        acc.at[pl.ds(s * WB + m * 104, 104)])

    @pl.when(s == NS - 1)
    def _():
        pltpu.sync_copy(rows0.at[pl.ds(0, N - NS * WB)],
                        acc.at[pl.ds(NS * WB, N - NS * WB)])
    plsc.subcore_barrier()

    pltpu.sync_copy(src_hbm.at[pl.ds(s * K, K)], src_idx)
    pltpu.sync_copy(dst_hbm.at[pl.ds(s * K, K)], dst_idx)

    def gather(j, b):
        pltpu.async_copy(xt_sc.at[src_idx.at[j]], rows[b], gsem[b])

    for j in range(NB - 1):
        gather(j, j)

    def group(g, _):
        for b in range(NB):
            j = g * NB + b
            bn = (b + NB - 1) % NB

            @pl.when(j >= 1)
            def _():
                pltpu.make_async_copy(rows[bn], acc.at[dst_idx.at[j]],
                                      ssem[bn]).wait()

            @pl.when(j + NB - 1 < K)
            def _():
                gather(j + NB - 1, bn)
            pltpu.make_async_copy(xt_sc.at[src_idx.at[j]], rows[b],
                                  gsem[b]).wait()
            pltpu.async_copy(rows[b], acc.at[dst_idx.at[j]], ssem[b],
                             add=True)
        return 0
    lax.fori_loop(0, K // NB, group, 0)

    pltpu.make_async_copy(rows[(K - 1) % NB], acc.at[dst_idx.at[K - 1]],
                          ssem[(K - 1) % NB]).wait()

    plsc.subcore_barrier()
    pltpu.sync_copy(acc.at[pl.ds(s * WB, WB)],
                    out_hbm.at[c, pl.ds(s * WB, WB)])

    @pl.when(s == NS - 1)
    def _():
        pltpu.sync_copy(acc.at[pl.ds(NS * WB, N - NS * WB)],
                        out_hbm.at[c, pl.ds(NS * WB, N - NS * WB)])


def _aggregate(xt, src, dst):
    mesh = plsc.VectorSubcoreMesh(core_axis_name="c", subcore_axis_name="s")
    f = pl.kernel(
        _agg_body,
        mesh=mesh,
        compiler_params=pltpu.CompilerParams(use_tc_tiling_on_sc=False),
        out_type=jax.ShapeDtypeStruct((NC, N, DC), jnp.float32),
        scratch_types=[
            pltpu.VMEM_SHARED((N, DC), jnp.float32),
            pltpu.VMEM((K, CH), jnp.int32),
            pltpu.VMEM((K, CH), jnp.int32),
            pltpu.VMEM((CH, DC), jnp.float32),
            pltpu.VMEM((CH, DC), jnp.float32),
            pltpu.VMEM((CH, DC), jnp.float32),
            pltpu.VMEM((CH, DC), jnp.float32),
            pltpu.SemaphoreType.DMA,
            pltpu.SemaphoreType.DMA,
            pltpu.SemaphoreType.DMA,
            pltpu.SemaphoreType.DMA,
            pltpu.SemaphoreType.DMA,
            pltpu.SemaphoreType.DMA,
            pltpu.SemaphoreType.DMA,
            pltpu.SemaphoreType.DMA,
        ],
    )
    return f(xt, src, dst)


def kernel(x, edge_index, W):
    src = edge_index[0].reshape(E // CH, CH)
    dst = edge_index[1].reshape(E // CH, CH)
    xt1 = _linear(x, W)
    p = _aggregate(xt1, src, dst)
    xt2 = _mid(p)
    q = _aggregate(xt2, src, dst)
    return _final(q)
